# baseline (speedup 1.0000x reference)
import functools

import jax
import jax.numpy as jnp
from jax import lax
from jax.experimental import pallas as pl
from jax.experimental.pallas import tpu as pltpu
from jax.experimental.pallas import tpu_sc as plsc

_N = 10000
_E = 320000
_G = 8
_D = 16

_CH = 125
_ROWS = _E // _CH
_RPT = _ROWS // 32
_NSL = _N // 16

_mesh = plsc.VectorSubcoreMesh(core_axis_name="c", subcore_axis_name="s")
_sc_params = pltpu.CompilerParams(use_tc_tiling_on_sc=False,
                                  needs_layout_passes=False)


def _fill_rows(buf, val):
    @pl.loop(0, _CH)
    def _(r):
        buf[r, pl.ds(0, _D)] = jnp.full((_D,), val, jnp.float32)


def _zero_acc(zbuf, acc, s):
    @pl.loop(0, _NSL // _CH)
    def _(kk):
        pltpu.sync_copy(zbuf, acc.at[pl.ds(s * _NSL + kk * _CH, _CH)])


def _copy_out(acc, out_h, c, s):
    off = pl.multiple_of(jnp.minimum(s * 632, _N - 632), 8)
    pltpu.sync_copy(acc.at[pl.ds(off, 632)], out_h.at[c, pl.ds(off, 632)])


def _rsqrt16(d):
    i = plsc.bitcast(d, jnp.int32)
    i = jnp.int32(0x5F3759DF) - lax.shift_right_logical(i, 1)
    y = plsc.bitcast(i, jnp.float32)
    for _ in range(3):
        y = y * (1.5 - 0.5 * d * y * y)
    return y


def _agg_pipeline(table_sp, acc, sidx, didx, rows, gsem, ssem):
    def g_start(j, b):
        pltpu.async_copy(table_sp.at[sidx.at[j]], rows[b], gsem[b])

    def g_wait(j, b):
        pltpu.make_async_copy(table_sp.at[sidx.at[j]], rows[b],
                              gsem[b]).wait()

    def s_start(j, b):
        pltpu.async_copy(rows[b], acc.at[didx.at[j]], ssem[b], add=True)

    def s_wait(j, b):
        pltpu.make_async_copy(rows[b], acc.at[didx.at[j]], ssem[b]).wait()

    for b in range(4):
        g_start(b, b)

    @pl.loop(0, _RPT - 4, step=4)
    def _(j):
        for b in range(4):
            g_wait(j + b, b)
            s_start(j + b, b)
        for b in range(4):
            s_wait(j + b, b)
            g_start(j + 4 + b, b)

    for b in range(4):
        g_wait(_RPT - 4 + b, b)
        s_start(_RPT - 4 + b, b)
    for b in range(4):
        s_wait(_RPT - 4 + b, b)


def _sc_deg(e3):
    @functools.partial(
        pl.kernel,
        out_type=[
            jax.ShapeDtypeStruct((2, _N, _D), jnp.float32),
            jax.ShapeDtypeStruct((_ROWS, _CH), jnp.int32),
            jax.ShapeDtypeStruct((_ROWS, _CH), jnp.int32),
        ],
        mesh=_mesh,
        compiler_params=_sc_params,
        scratch_types=[
            pltpu.VMEM((_RPT, _CH), jnp.int32),
            pltpu.VMEM((_RPT, _CH), jnp.int32),
            pltpu.VMEM((_CH, _D), jnp.float32),
            pltpu.VMEM_SHARED((_N, _D), jnp.float32),
            pltpu.SemaphoreType.DMA,
        ],
    )
    def k(e_h, hist_h, srco_h, dsto_h, sidx, didx, ones, acc, sem):
        c = lax.axis_index("c")
        s = lax.axis_index("s")
        w = c * 16 + s
        _fill_rows(ones, 0.0)
        _zero_acc(ones, acc, s)
        _fill_rows(ones, 1.0)
        pltpu.sync_copy(e_h.at[0, pl.ds(w * _RPT, _RPT)], sidx)
        pltpu.sync_copy(e_h.at[1, pl.ds(w * _RPT, _RPT)], didx)
        pltpu.sync_copy(sidx, srco_h.at[pl.ds(w * _RPT, _RPT)])
        pltpu.sync_copy(didx, dsto_h.at[pl.ds(w * _RPT, _RPT)])
        plsc.subcore_barrier()

        @pl.loop(0, 8)
        def _(j):
            pltpu.async_copy(ones, acc.at[didx.at[j]], sem, add=True)

        @pl.loop(8, _RPT, step=8)
        def _(j):
            @pl.loop(0, 8)
            def _(b):
                pltpu.async_copy(ones, acc.at[didx.at[j + b]], sem, add=True)
            @pl.loop(0, 8)
            def _(b):
                pltpu.make_async_copy(ones, acc.at[didx.at[j - 8 + b]],
                                      sem).wait()

        @pl.loop(_RPT - 8, _RPT)
        def _(j):
            pltpu.make_async_copy(ones, acc.at[didx.at[j]], sem).wait()

        plsc.subcore_barrier()
        _copy_out(acc, hist_h, c, s)

    return k(e3)


def _agg_scratch(n_prologue_bufs):
    return (
        [
            pltpu.VMEM((_RPT, _CH), jnp.int32),
            pltpu.VMEM((_RPT, _CH), jnp.int32),
            pltpu.VMEM((_CH, _D), jnp.float32),
            pltpu.VMEM((_CH, _D), jnp.float32),
            pltpu.VMEM((_CH, _D), jnp.float32),
            pltpu.VMEM((_CH, _D), jnp.float32),
            pltpu.VMEM((_CH, _D), jnp.float32),
        ]
        + [pltpu.VMEM((_NSL, _D), jnp.float32)
           for _ in range(n_prologue_bufs)]
        + [pltpu.VMEM_SHARED((_N, _D), jnp.float32)]
        + [pltpu.SemaphoreType.DMA for _ in range(8)]
    )


def _sc_agg1(src_sc, dst_sc, hist, xw):
    @functools.partial(
        pl.kernel,
        out_type=[
            jax.ShapeDtypeStruct((2, _N, _D), jnp.float32),
            jax.ShapeDtypeStruct((_N, _D), jnp.float32),
        ],
        mesh=_mesh,
        compiler_params=_sc_params,
        scratch_types=_agg_scratch(3),
    )
    def k(src_h, dst_h, hist_h, xw_h, out_h, table, sidx, didx, zbuf,
          r0, r1, r2, r3, h0b, h1b, xwb, acc,
          g0, g1, g2, g3, s0, s1, s2, s3):
        c = lax.axis_index("c")
        s = lax.axis_index("s")
        w = c * 16 + s
        base = s * _NSL
        pltpu.sync_copy(hist_h.at[0, pl.ds(base, _NSL)], h0b)
        pltpu.sync_copy(hist_h.at[1, pl.ds(base, _NSL)], h1b)
        pltpu.sync_copy(xw_h.at[pl.ds(base, _NSL)], xwb)
        pltpu.sync_copy(src_h.at[pl.ds(w * _RPT, _RPT)], sidx)
        pltpu.sync_copy(dst_h.at[pl.ds(w * _RPT, _RPT)], didx)

        @pl.loop(0, _NSL)
        def _(r):
            d16 = h0b[r, pl.ds(0, _D)] + h1b[r, pl.ds(0, _D)] + 1.0
            y = _rsqrt16(d16)
            xwb[r, pl.ds(0, _D)] = y * xwb[r, pl.ds(0, _D)]

        pltpu.sync_copy(xwb, table.at[pl.ds(base, _NSL)])
        _fill_rows(zbuf, 0.0)
        _zero_acc(zbuf, acc, s)
        plsc.subcore_barrier()
        _agg_pipeline(table, acc, sidx, didx, (r0, r1, r2, r3),
                      (g0, g1, g2, g3), (s0, s1, s2, s3))
        plsc.subcore_barrier()
        _copy_out(acc, out_h, c, s)

    return k(src_sc, dst_sc, hist, xw)


def _sc_agg2(src_sc, dst_sc, hist, xw, part1, b1row):
    @functools.partial(
        pl.kernel,
        out_type=[
            jax.ShapeDtypeStruct((2, _N, _D), jnp.float32),
            jax.ShapeDtypeStruct((_N, _D), jnp.float32),
        ],
        mesh=_mesh,
        compiler_params=_sc_params,
        scratch_types=_agg_scratch(5) + [pltpu.VMEM((1, _D), jnp.float32)],
    )
    def k(src_h, dst_h, hist_h, xw_h, p1_h, b1_h, out_h, table, sidx, didx,
          zbuf, r0, r1, r2, r3, h0b, h1b, xwb, p0b, p1b, acc,
          g0, g1, g2, g3, s0, s1, s2, s3, b1b):
        c = lax.axis_index("c")
        s = lax.axis_index("s")
        w = c * 16 + s
        base = s * _NSL
        pltpu.sync_copy(hist_h.at[0, pl.ds(base, _NSL)], h0b)
        pltpu.sync_copy(hist_h.at[1, pl.ds(base, _NSL)], h1b)
        pltpu.sync_copy(xw_h.at[pl.ds(base, _NSL)], xwb)
        pltpu.sync_copy(p1_h.at[0, pl.ds(base, _NSL)], p0b)
        pltpu.sync_copy(p1_h.at[1, pl.ds(base, _NSL)], p1b)
        pltpu.sync_copy(b1_h, b1b)
        pltpu.sync_copy(src_h.at[pl.ds(w * _RPT, _RPT)], sidx)
        pltpu.sync_copy(dst_h.at[pl.ds(w * _RPT, _RPT)], didx)
        b1v = b1b[0, pl.ds(0, _D)]

        @pl.loop(0, _NSL)
        def _(r):
            d16 = h0b[r, pl.ds(0, _D)] + h1b[r, pl.ds(0, _D)] + 1.0
            y = _rsqrt16(d16)
            u1 = y * xwb[r, pl.ds(0, _D)]
            h1 = y * (p0b[r, pl.ds(0, _D)] + p1b[r, pl.ds(0, _D)] + u1) + b1v
            xwb[r, pl.ds(0, _D)] = y * jnp.maximum(h1, 0.0)

        pltpu.sync_copy(xwb, table.at[pl.ds(base, _NSL)])
        _fill_rows(zbuf, 0.0)
        _zero_acc(zbuf, acc, s)
        plsc.subcore_barrier()
        _agg_pipeline(table, acc, sidx, didx, (r0, r1, r2, r3),
                      (g0, g1, g2, g3), (s0, s1, s2, s3))
        plsc.subcore_barrier()
        pltpu.sync_copy(acc.at[pl.ds(base, _NSL)], p0b)

        @pl.loop(0, _NSL)
        def _(r):
            d16 = h0b[r, pl.ds(0, _D)] + h1b[r, pl.ds(0, _D)] + 1.0
            y = _rsqrt16(d16)
            p0b[r, pl.ds(0, _D)] = y * (p0b[r, pl.ds(0, _D)]
                                        + 0.5 * xwb[r, pl.ds(0, _D)])

        pltpu.sync_copy(p0b, acc.at[pl.ds(base, _NSL)])
        plsc.subcore_barrier()
        _copy_out(acc, out_h, c, s)

    return k(src_sc, dst_sc, hist, xw, part1, b1row)


def _tc_xw_body(x_ref, w_ref, o_ref):
    o_ref[...] = jnp.dot(x_ref[...], w_ref[...],
                         preferred_element_type=jnp.float32)


def _tc_xw(x, W1):
    f_in, h1 = W1.shape
    return pl.pallas_call(
        _tc_xw_body,
        out_shape=jax.ShapeDtypeStruct((_N, h1), jnp.float32),
    )(x, W1)


def _tc_c_body(p2_ref, w2_ref, b2_ref,
               bat_ref, l1w_ref, l1b_ref, l2w_ref, l2b_ref, out_ref):
    t = p2_ref[0] + p2_ref[1]
    h = jnp.dot(t, w2_ref[...],
                preferred_element_type=jnp.float32) + b2_ref[...]
    h = jnp.maximum(h, 0.0)
    hc = jnp.concatenate([h, jnp.ones((_N, 1), jnp.float32)], axis=1)
    onehot = (bat_ref[...] == lax.broadcasted_iota(jnp.int32, (1, _G), 1))
    m = onehot.astype(jnp.float32)
    sums = lax.dot_general(m, hc, (((0,), (0,)), ((), ())),
                           preferred_element_type=jnp.float32)
    h2 = sums.shape[1] - 1
    cnt = sums[:, h2:h2 + 1]
    pooled = sums[:, 0:h2] / jnp.maximum(cnt, 1.0)
    z = jnp.dot(pooled, l1w_ref[...],
                preferred_element_type=jnp.float32) + l1b_ref[...]
    z = jnp.maximum(z, 0.0)
    z = jnp.dot(z, l2w_ref[...],
                preferred_element_type=jnp.float32) + l2b_ref[...]
    mx = jnp.max(z, axis=1, keepdims=True)
    lse = mx + jnp.log(jnp.sum(jnp.exp(z - mx), axis=1, keepdims=True))
    out_ref[...] = z - lse


def _tc_c(part2, W2, b2row, batch2, L1W, L1b, L2W, L2b):
    c = L2W.shape[1]
    return pl.pallas_call(
        _tc_c_body,
        out_shape=jax.ShapeDtypeStruct((_G, c), jnp.float32),
    )(part2, W2, b2row, batch2, L1W, L1b, L2W, L2b)


def kernel(x, edge_index, batch, W1, b1, W2, b2, L1W, L1b, L2W, L2b):
    e3 = edge_index.reshape(2, _ROWS, _CH)
    batch2 = batch.reshape(_N, 1)
    b1row = b1.reshape(1, -1)

    hist, src_sc, dst_sc = _sc_deg(e3)
    xw = _tc_xw(x, W1)
    part1, _tab1 = _sc_agg1(src_sc, dst_sc, hist, xw)
    part2, _tab2 = _sc_agg2(src_sc, dst_sc, hist, xw, part1, b1row)
    return _tc_c(part2, W2, b2.reshape(1, -1), batch2,
                 L1W, L1b.reshape(1, -1), L2W, L2b.reshape(1, -1))


# speedup vs baseline: 1.0923x; 1.0923x over previous
import functools

import jax
import jax.numpy as jnp
from jax import lax
from jax.experimental import pallas as pl
from jax.experimental.pallas import tpu as pltpu
from jax.experimental.pallas import tpu_sc as plsc

_N = 10000
_E = 320000
_G = 8
_D = 16

_CH = 125
_ROWS = _E // _CH
_RPT = _ROWS // 32
_NSL = _N // 16

_mesh = plsc.VectorSubcoreMesh(core_axis_name="c", subcore_axis_name="s")
_sc_params = pltpu.CompilerParams(use_tc_tiling_on_sc=False,
                                  needs_layout_passes=False)


def _fill_rows(buf, val):
    @pl.loop(0, _CH)
    def _(r):
        buf[r, pl.ds(0, _D)] = jnp.full((_D,), val, jnp.float32)


def _zero_acc(zbuf, acc, s):
    @pl.loop(0, _NSL // _CH)
    def _(kk):
        pltpu.sync_copy(zbuf, acc.at[pl.ds(s * _NSL + kk * _CH, _CH)])


def _copy_out(acc, out_h, c, s):
    off = pl.multiple_of(jnp.minimum(s * 632, _N - 632), 8)
    pltpu.sync_copy(acc.at[pl.ds(off, 632)], out_h.at[c, pl.ds(off, 632)])


def _rsqrt16(d):
    i = plsc.bitcast(d, jnp.int32)
    i = jnp.int32(0x5F3759DF) - lax.shift_right_logical(i, 1)
    y = plsc.bitcast(i, jnp.float32)
    for _ in range(3):
        y = y * (1.5 - 0.5 * d * y * y)
    return y


def _agg_pipeline(table_sp, acc, sidx, didx, rows, gsem, ssem):
    def g_start(j, b):
        pltpu.async_copy(table_sp.at[sidx.at[j]], rows[b], gsem[b])

    def g_wait(j, b):
        pltpu.make_async_copy(table_sp.at[sidx.at[j]], rows[b],
                              gsem[b]).wait()

    def s_start(j, b):
        pltpu.async_copy(rows[b], acc.at[didx.at[j]], ssem[b], add=True)

    def s_wait(j, b):
        pltpu.make_async_copy(rows[b], acc.at[didx.at[j]], ssem[b]).wait()

    for b in range(4):
        g_start(b, b)

    @pl.loop(0, _RPT - 4, step=4)
    def _(j):
        for b in range(4):
            g_wait(j + b, b)
            s_start(j + b, b)
        for b in range(4):
            s_wait(j + b, b)
            g_start(j + 4 + b, b)

    for b in range(4):
        g_wait(_RPT - 4 + b, b)
        s_start(_RPT - 4 + b, b)
    for b in range(4):
        s_wait(_RPT - 4 + b, b)


def _sc_deg(e3):
    @functools.partial(
        pl.kernel,
        out_type=[
            jax.ShapeDtypeStruct((2, _N, _D), jnp.float32),
            jax.ShapeDtypeStruct((_ROWS, _CH), jnp.int32),
            jax.ShapeDtypeStruct((_ROWS, _CH), jnp.int32),
        ],
        mesh=_mesh,
        compiler_params=_sc_params,
        scratch_types=[
            pltpu.VMEM((_RPT, _CH), jnp.int32),
            pltpu.VMEM((_RPT, _CH), jnp.int32),
            pltpu.VMEM((_CH, _D), jnp.float32),
            pltpu.VMEM_SHARED((_N, _D), jnp.float32),
            pltpu.SemaphoreType.DMA,
        ],
    )
    def k(e_h, hist_h, srco_h, dsto_h, sidx, didx, ones, acc, sem):
        c = lax.axis_index("c")
        s = lax.axis_index("s")
        w = c * 16 + s
        _fill_rows(ones, 0.0)
        _zero_acc(ones, acc, s)
        _fill_rows(ones, 1.0)
        pltpu.sync_copy(e_h.at[0, pl.ds(w * _RPT, _RPT)], sidx)
        pltpu.sync_copy(e_h.at[1, pl.ds(w * _RPT, _RPT)], didx)
        pltpu.sync_copy(sidx, srco_h.at[pl.ds(w * _RPT, _RPT)])
        pltpu.sync_copy(didx, dsto_h.at[pl.ds(w * _RPT, _RPT)])
        plsc.subcore_barrier()

        @pl.loop(0, 8)
        def _(j):
            pltpu.async_copy(ones, acc.at[didx.at[j]], sem, add=True)

        @pl.loop(8, _RPT, step=8)
        def _(j):
            @pl.loop(0, 8)
            def _(b):
                pltpu.async_copy(ones, acc.at[didx.at[j + b]], sem, add=True)
            @pl.loop(0, 8)
            def _(b):
                pltpu.make_async_copy(ones, acc.at[didx.at[j - 8 + b]],
                                      sem).wait()

        @pl.loop(_RPT - 8, _RPT)
        def _(j):
            pltpu.make_async_copy(ones, acc.at[didx.at[j]], sem).wait()

        plsc.subcore_barrier()
        _copy_out(acc, hist_h, c, s)

    return k(e3)


def _agg_scratch(n_prologue_bufs):
    return (
        [
            pltpu.VMEM((_RPT, _CH), jnp.int32),
            pltpu.VMEM((_RPT, _CH), jnp.int32),
            pltpu.VMEM((_CH, _D), jnp.float32),
            pltpu.VMEM((_CH, _D), jnp.float32),
            pltpu.VMEM((_CH, _D), jnp.float32),
            pltpu.VMEM((_CH, _D), jnp.float32),
            pltpu.VMEM((_CH, _D), jnp.float32),
        ]
        + [pltpu.VMEM((_NSL, _D), jnp.float32)
           for _ in range(n_prologue_bufs)]
        + [
            pltpu.VMEM_SHARED((_N, _D), jnp.float32),
            pltpu.VMEM_SHARED((_N, _D), jnp.float32),
        ]
        + [pltpu.SemaphoreType.DMA for _ in range(8)]
    )


def _sc_agg1(src_sc, dst_sc, hist, xw):
    @functools.partial(
        pl.kernel,
        out_type=jax.ShapeDtypeStruct((2, _N, _D), jnp.float32),
        mesh=_mesh,
        compiler_params=_sc_params,
        scratch_types=_agg_scratch(3),
    )
    def k(src_h, dst_h, hist_h, xw_h, out_h, sidx, didx, zbuf,
          r0, r1, r2, r3, h0b, h1b, xwb, table, acc,
          g0, g1, g2, g3, s0, s1, s2, s3):
        c = lax.axis_index("c")
        s = lax.axis_index("s")
        w = c * 16 + s
        base = s * _NSL
        pltpu.sync_copy(hist_h.at[0, pl.ds(base, _NSL)], h0b)
        pltpu.sync_copy(hist_h.at[1, pl.ds(base, _NSL)], h1b)
        pltpu.sync_copy(xw_h.at[pl.ds(base, _NSL)], xwb)
        pltpu.sync_copy(src_h.at[pl.ds(w * _RPT, _RPT)], sidx)
        pltpu.sync_copy(dst_h.at[pl.ds(w * _RPT, _RPT)], didx)

        @pl.loop(0, _NSL)
        def _(r):
            d16 = h0b[r, pl.ds(0, _D)] + h1b[r, pl.ds(0, _D)] + 1.0
            y = _rsqrt16(d16)
            xwb[r, pl.ds(0, _D)] = y * xwb[r, pl.ds(0, _D)]

        pltpu.sync_copy(xwb, table.at[pl.ds(base, _NSL)])
        _fill_rows(zbuf, 0.0)
        _zero_acc(zbuf, acc, s)
        plsc.subcore_barrier()
        _agg_pipeline(table, acc, sidx, didx, (r0, r1, r2, r3),
                      (g0, g1, g2, g3), (s0, s1, s2, s3))
        plsc.subcore_barrier()
        _copy_out(acc, out_h, c, s)

    return k(src_sc, dst_sc, hist, xw)


def _sc_agg2(src_sc, dst_sc, hist, xw, part1, b1row):
    @functools.partial(
        pl.kernel,
        out_type=jax.ShapeDtypeStruct((2, _N, _D), jnp.float32),
        mesh=_mesh,
        compiler_params=_sc_params,
        scratch_types=_agg_scratch(5) + [pltpu.VMEM((1, _D), jnp.float32)],
    )
    def k(src_h, dst_h, hist_h, xw_h, p1_h, b1_h, out_h, sidx, didx,
          zbuf, r0, r1, r2, r3, h0b, h1b, xwb, p0b, p1b, table, acc,
          g0, g1, g2, g3, s0, s1, s2, s3, b1b):
        c = lax.axis_index("c")
        s = lax.axis_index("s")
        w = c * 16 + s
        base = s * _NSL
        pltpu.sync_copy(hist_h.at[0, pl.ds(base, _NSL)], h0b)
        pltpu.sync_copy(hist_h.at[1, pl.ds(base, _NSL)], h1b)
        pltpu.sync_copy(xw_h.at[pl.ds(base, _NSL)], xwb)
        pltpu.sync_copy(p1_h.at[0, pl.ds(base, _NSL)], p0b)
        pltpu.sync_copy(p1_h.at[1, pl.ds(base, _NSL)], p1b)
        pltpu.sync_copy(b1_h, b1b)
        pltpu.sync_copy(src_h.at[pl.ds(w * _RPT, _RPT)], sidx)
        pltpu.sync_copy(dst_h.at[pl.ds(w * _RPT, _RPT)], didx)
        b1v = b1b[0, pl.ds(0, _D)]

        @pl.loop(0, _NSL)
        def _(r):
            d16 = h0b[r, pl.ds(0, _D)] + h1b[r, pl.ds(0, _D)] + 1.0
            y = _rsqrt16(d16)
            u1 = y * xwb[r, pl.ds(0, _D)]
            h1 = y * (p0b[r, pl.ds(0, _D)] + p1b[r, pl.ds(0, _D)] + u1) + b1v
            xwb[r, pl.ds(0, _D)] = y * jnp.maximum(h1, 0.0)

        pltpu.sync_copy(xwb, table.at[pl.ds(base, _NSL)])
        _fill_rows(zbuf, 0.0)
        _zero_acc(zbuf, acc, s)
        plsc.subcore_barrier()
        _agg_pipeline(table, acc, sidx, didx, (r0, r1, r2, r3),
                      (g0, g1, g2, g3), (s0, s1, s2, s3))
        plsc.subcore_barrier()
        pltpu.sync_copy(acc.at[pl.ds(base, _NSL)], p0b)

        @pl.loop(0, _NSL)
        def _(r):
            d16 = h0b[r, pl.ds(0, _D)] + h1b[r, pl.ds(0, _D)] + 1.0
            y = _rsqrt16(d16)
            p0b[r, pl.ds(0, _D)] = y * (p0b[r, pl.ds(0, _D)]
                                        + 0.5 * xwb[r, pl.ds(0, _D)])

        pltpu.sync_copy(p0b, acc.at[pl.ds(base, _NSL)])
        plsc.subcore_barrier()
        _copy_out(acc, out_h, c, s)

    return k(src_sc, dst_sc, hist, xw, part1, b1row)


def _tc_xw_body(x_ref, w_ref, o_ref):
    o_ref[...] = jnp.dot(x_ref[...], w_ref[...],
                         preferred_element_type=jnp.float32)


def _tc_xw(x, W1):
    f_in, h1 = W1.shape
    return pl.pallas_call(
        _tc_xw_body,
        out_shape=jax.ShapeDtypeStruct((_N, h1), jnp.float32),
    )(x, W1)


def _tc_c_body(p2_ref, w2_ref, b2_ref,
               bat_ref, l1w_ref, l1b_ref, l2w_ref, l2b_ref, out_ref):
    t = p2_ref[0] + p2_ref[1]
    h = jnp.dot(t, w2_ref[...],
                preferred_element_type=jnp.float32) + b2_ref[...]
    h = jnp.maximum(h, 0.0)
    hc = jnp.concatenate([h, jnp.ones((_N, 1), jnp.float32)], axis=1)
    onehot = (bat_ref[...] == lax.broadcasted_iota(jnp.int32, (_G, 1), 0))
    m = onehot.astype(jnp.float32)
    sums = lax.dot_general(m, hc, (((1,), (0,)), ((), ())),
                           preferred_element_type=jnp.float32)
    h2 = sums.shape[1] - 1
    cnt = sums[:, h2:h2 + 1]
    pooled = sums[:, 0:h2] / jnp.maximum(cnt, 1.0)
    z = jnp.dot(pooled, l1w_ref[...],
                preferred_element_type=jnp.float32) + l1b_ref[...]
    z = jnp.maximum(z, 0.0)
    z = jnp.dot(z, l2w_ref[...],
                preferred_element_type=jnp.float32) + l2b_ref[...]
    mx = jnp.max(z, axis=1, keepdims=True)
    lse = mx + jnp.log(jnp.sum(jnp.exp(z - mx), axis=1, keepdims=True))
    out_ref[...] = z - lse


def _tc_c(part2, W2, b2row, batch2, L1W, L1b, L2W, L2b):
    c = L2W.shape[1]
    return pl.pallas_call(
        _tc_c_body,
        out_shape=jax.ShapeDtypeStruct((_G, c), jnp.float32),
    )(part2, W2, b2row, batch2, L1W, L1b, L2W, L2b)


def kernel(x, edge_index, batch, W1, b1, W2, b2, L1W, L1b, L2W, L2b):
    e3 = edge_index.reshape(2, _ROWS, _CH)
    batch2 = batch.reshape(1, _N)
    b1row = b1.reshape(1, -1)

    hist, src_sc, dst_sc = _sc_deg(e3)
    xw = _tc_xw(x, W1)
    part1 = _sc_agg1(src_sc, dst_sc, hist, xw)
    part2 = _sc_agg2(src_sc, dst_sc, hist, xw, part1, b1row)
    return _tc_c(part2, W2, b2.reshape(1, -1), batch2,
                 L1W, L1b.reshape(1, -1), L2W, L2b.reshape(1, -1))


# speedup vs baseline: 1.1366x; 1.0405x over previous
import functools

import jax
import jax.numpy as jnp
from jax import lax
from jax.experimental import pallas as pl
from jax.experimental.pallas import tpu as pltpu
from jax.experimental.pallas import tpu_sc as plsc

_N = 10000
_E = 320000
_G = 8
_D = 16

_CH = 125
_ROWS = _E // _CH
_RPT = _ROWS // 32
_NSL = _N // 16

_mesh = plsc.VectorSubcoreMesh(core_axis_name="c", subcore_axis_name="s")
_sc_params = pltpu.CompilerParams(use_tc_tiling_on_sc=False,
                                  needs_layout_passes=False)


def _fill_rows(buf, val):
    @pl.loop(0, _CH)
    def _(r):
        buf[r, pl.ds(0, _D)] = jnp.full((_D,), val, jnp.float32)


def _zero_acc(zbuf, acc, s):
    @pl.loop(0, _NSL // _CH)
    def _(kk):
        pltpu.sync_copy(zbuf, acc.at[pl.ds(s * _NSL + kk * _CH, _CH)])


def _copy_out(acc, out_h, c, s):
    off = pl.multiple_of(jnp.minimum(s * 632, _N - 632), 8)
    pltpu.sync_copy(acc.at[pl.ds(off, 632)], out_h.at[c, pl.ds(off, 632)])


def _rsqrt16(d):
    i = plsc.bitcast(d, jnp.int32)
    i = jnp.int32(0x5F3759DF) - lax.shift_right_logical(i, 1)
    y = plsc.bitcast(i, jnp.float32)
    for _ in range(3):
        y = y * (1.5 - 0.5 * d * y * y)
    return y


def _agg_pipeline(table_sp, acc, sidx, didx, rows, gsem, ssem):
    def g_start(j, b):
        pltpu.async_copy(table_sp.at[sidx.at[j]], rows[b], gsem[b])

    def g_wait(j, b):
        pltpu.make_async_copy(table_sp.at[sidx.at[j]], rows[b],
                              gsem[b]).wait()

    def s_start(j, b):
        pltpu.async_copy(rows[b], acc.at[didx.at[j]], ssem[b], add=True)

    def s_wait(j, b):
        pltpu.make_async_copy(rows[b], acc.at[didx.at[j]], ssem[b]).wait()

    for b in range(4):
        g_start(b, b)

    @pl.loop(0, _RPT - 4, step=4)
    def _(j):
        for b in range(4):
            g_wait(j + b, b)
            s_start(j + b, b)
        for b in range(4):
            s_wait(j + b, b)
            g_start(j + 4 + b, b)

    for b in range(4):
        g_wait(_RPT - 4 + b, b)
        s_start(_RPT - 4 + b, b)
    for b in range(4):
        s_wait(_RPT - 4 + b, b)


def _sc_deg(e3):
    @functools.partial(
        pl.kernel,
        out_type=[
            jax.ShapeDtypeStruct((2, _N, _D), jnp.float32),
            jax.ShapeDtypeStruct((_ROWS, _CH), jnp.int32),
            jax.ShapeDtypeStruct((_ROWS, _CH), jnp.int32),
        ],
        mesh=_mesh,
        compiler_params=_sc_params,
        scratch_types=[
            pltpu.VMEM((_RPT, _CH), jnp.int32),
            pltpu.VMEM((_RPT, _CH), jnp.int32),
            pltpu.VMEM((_CH, _D), jnp.float32),
            pltpu.VMEM_SHARED((_N, _D), jnp.float32),
            pltpu.SemaphoreType.DMA,
        ],
    )
    def k(e_h, hist_h, srco_h, dsto_h, sidx, didx, ones, acc, sem):
        c = lax.axis_index("c")
        s = lax.axis_index("s")
        w = c * 16 + s
        _fill_rows(ones, 0.0)
        _zero_acc(ones, acc, s)
        _fill_rows(ones, 1.0)
        pltpu.sync_copy(e_h.at[0, pl.ds(w * _RPT, _RPT)], sidx)
        pltpu.sync_copy(e_h.at[1, pl.ds(w * _RPT, _RPT)], didx)
        pltpu.sync_copy(sidx, srco_h.at[pl.ds(w * _RPT, _RPT)])
        pltpu.sync_copy(didx, dsto_h.at[pl.ds(w * _RPT, _RPT)])
        plsc.subcore_barrier()

        @pl.loop(0, 8)
        def _(j):
            pltpu.async_copy(ones, acc.at[didx.at[j]], sem, add=True)

        @pl.loop(8, _RPT, step=8)
        def _(j):
            @pl.loop(0, 8)
            def _(b):
                pltpu.async_copy(ones, acc.at[didx.at[j + b]], sem, add=True)
            @pl.loop(0, 8)
            def _(b):
                pltpu.make_async_copy(ones, acc.at[didx.at[j - 8 + b]],
                                      sem).wait()

        @pl.loop(_RPT - 8, _RPT)
        def _(j):
            pltpu.make_async_copy(ones, acc.at[didx.at[j]], sem).wait()

        plsc.subcore_barrier()
        _copy_out(acc, hist_h, c, s)

    return k(e3)


def _agg_scratch(n_prologue_bufs):
    return (
        [
            pltpu.VMEM((_RPT, _CH), jnp.int32),
            pltpu.VMEM((_RPT, _CH), jnp.int32),
            pltpu.VMEM((_CH, _D), jnp.float32),
            pltpu.VMEM((_CH, _D), jnp.float32),
            pltpu.VMEM((_CH, _D), jnp.float32),
            pltpu.VMEM((_CH, _D), jnp.float32),
            pltpu.VMEM((_CH, _D), jnp.float32),
        ]
        + [pltpu.VMEM((_NSL, _D), jnp.float32)
           for _ in range(n_prologue_bufs)]
        + [
            pltpu.VMEM_SHARED((_N, _D), jnp.float32),
            pltpu.VMEM_SHARED((_N, _D), jnp.float32),
        ]
        + [pltpu.SemaphoreType.DMA for _ in range(8)]
    )


def _sc_agg1(src_sc, dst_sc, hist, xw):
    @functools.partial(
        pl.kernel,
        out_type=jax.ShapeDtypeStruct((2, _N, _D), jnp.float32),
        mesh=_mesh,
        compiler_params=_sc_params,
        scratch_types=_agg_scratch(3),
    )
    def k(src_h, dst_h, hist_h, xw_h, out_h, sidx, didx, zbuf,
          r0, r1, r2, r3, h0b, h1b, xwb, table, acc,
          g0, g1, g2, g3, s0, s1, s2, s3):
        c = lax.axis_index("c")
        s = lax.axis_index("s")
        w = c * 16 + s
        base = s * _NSL
        pltpu.sync_copy(hist_h.at[0, pl.ds(base, _NSL)], h0b)
        pltpu.sync_copy(hist_h.at[1, pl.ds(base, _NSL)], h1b)
        pltpu.sync_copy(xw_h.at[pl.ds(base, _NSL)], xwb)
        pltpu.sync_copy(src_h.at[pl.ds(w * _RPT, _RPT)], sidx)
        pltpu.sync_copy(dst_h.at[pl.ds(w * _RPT, _RPT)], didx)

        @pl.loop(0, _NSL, step=5)
        def _(r0_):
            for dd in range(5):
                r = r0_ + dd
                d16 = h0b[r, pl.ds(0, _D)] + h1b[r, pl.ds(0, _D)] + 1.0
                y = _rsqrt16(d16)
                xwb[r, pl.ds(0, _D)] = y * xwb[r, pl.ds(0, _D)]

        pltpu.sync_copy(xwb, table.at[pl.ds(base, _NSL)])
        _fill_rows(zbuf, 0.0)
        _zero_acc(zbuf, acc, s)
        plsc.subcore_barrier()
        _agg_pipeline(table, acc, sidx, didx, (r0, r1, r2, r3),
                      (g0, g1, g2, g3), (s0, s1, s2, s3))
        plsc.subcore_barrier()
        _copy_out(acc, out_h, c, s)

    return k(src_sc, dst_sc, hist, xw)


def _sc_agg2(src_sc, dst_sc, hist, xw, part1, b1row):
    @functools.partial(
        pl.kernel,
        out_type=jax.ShapeDtypeStruct((2, _N, _D), jnp.float32),
        mesh=_mesh,
        compiler_params=_sc_params,
        scratch_types=_agg_scratch(5) + [pltpu.VMEM((1, _D), jnp.float32)],
    )
    def k(src_h, dst_h, hist_h, xw_h, p1_h, b1_h, out_h, sidx, didx,
          zbuf, r0, r1, r2, r3, h0b, h1b, xwb, p0b, p1b, table, acc,
          g0, g1, g2, g3, s0, s1, s2, s3, b1b):
        c = lax.axis_index("c")
        s = lax.axis_index("s")
        w = c * 16 + s
        base = s * _NSL
        pltpu.sync_copy(hist_h.at[0, pl.ds(base, _NSL)], h0b)
        pltpu.sync_copy(hist_h.at[1, pl.ds(base, _NSL)], h1b)
        pltpu.sync_copy(xw_h.at[pl.ds(base, _NSL)], xwb)
        pltpu.sync_copy(p1_h.at[0, pl.ds(base, _NSL)], p0b)
        pltpu.sync_copy(p1_h.at[1, pl.ds(base, _NSL)], p1b)
        pltpu.sync_copy(b1_h, b1b)
        pltpu.sync_copy(src_h.at[pl.ds(w * _RPT, _RPT)], sidx)
        pltpu.sync_copy(dst_h.at[pl.ds(w * _RPT, _RPT)], didx)
        b1v = b1b[0, pl.ds(0, _D)]

        @pl.loop(0, _NSL, step=5)
        def _(r0_):
            for dd in range(5):
                r = r0_ + dd
                d16 = h0b[r, pl.ds(0, _D)] + h1b[r, pl.ds(0, _D)] + 1.0
                y = _rsqrt16(d16)
                u1 = y * xwb[r, pl.ds(0, _D)]
                h1 = (y * (p0b[r, pl.ds(0, _D)] + p1b[r, pl.ds(0, _D)] + u1)
                      + b1v)
                v = y * jnp.maximum(h1, 0.0)
                xwb[r, pl.ds(0, _D)] = v
                p1b[r, pl.ds(0, _D)] = 0.5 * v

        pltpu.sync_copy(xwb, table.at[pl.ds(base, _NSL)])
        pltpu.sync_copy(p1b, acc.at[pl.ds(base, _NSL)])
        plsc.subcore_barrier()
        _agg_pipeline(table, acc, sidx, didx, (r0, r1, r2, r3),
                      (g0, g1, g2, g3), (s0, s1, s2, s3))
        plsc.subcore_barrier()
        pltpu.sync_copy(acc.at[pl.ds(base, _NSL)], p0b)

        @pl.loop(0, _NSL, step=5)
        def _(r0_):
            for dd in range(5):
                r = r0_ + dd
                d16 = h0b[r, pl.ds(0, _D)] + h1b[r, pl.ds(0, _D)] + 1.0
                y = _rsqrt16(d16)
                p0b[r, pl.ds(0, _D)] = y * p0b[r, pl.ds(0, _D)]

        pltpu.sync_copy(p0b, acc.at[pl.ds(base, _NSL)])
        plsc.subcore_barrier()
        _copy_out(acc, out_h, c, s)

    return k(src_sc, dst_sc, hist, xw, part1, b1row)


def _tc_xw_body(x_ref, w_ref, o_ref):
    o_ref[...] = jnp.dot(x_ref[...], w_ref[...],
                         preferred_element_type=jnp.float32)


def _tc_xw(x, W1):
    f_in, h1 = W1.shape
    return pl.pallas_call(
        _tc_xw_body,
        out_shape=jax.ShapeDtypeStruct((_N, h1), jnp.float32),
    )(x, W1)


def _tc_c_body(p2_ref, w2_ref, b2_ref,
               bat_ref, l1w_ref, l1b_ref, l2w_ref, l2b_ref, out_ref):
    t = p2_ref[0] + p2_ref[1]
    h = jnp.dot(t, w2_ref[...],
                preferred_element_type=jnp.float32) + b2_ref[...]
    h = jnp.maximum(h, 0.0)
    hc = jnp.concatenate([h, jnp.ones((_N, 1), jnp.float32)], axis=1)
    onehot = (bat_ref[...] == lax.broadcasted_iota(jnp.int32, (_G, 1), 0))
    m = onehot.astype(jnp.float32)
    sums = lax.dot_general(m, hc, (((1,), (0,)), ((), ())),
                           preferred_element_type=jnp.float32)
    h2 = sums.shape[1] - 1
    cnt = sums[:, h2:h2 + 1]
    pooled = sums[:, 0:h2] / jnp.maximum(cnt, 1.0)
    z = jnp.dot(pooled, l1w_ref[...],
                preferred_element_type=jnp.float32) + l1b_ref[...]
    z = jnp.maximum(z, 0.0)
    z = jnp.dot(z, l2w_ref[...],
                preferred_element_type=jnp.float32) + l2b_ref[...]
    mx = jnp.max(z, axis=1, keepdims=True)
    lse = mx + jnp.log(jnp.sum(jnp.exp(z - mx), axis=1, keepdims=True))
    out_ref[...] = z - lse


def _tc_c(part2, W2, b2row, batch2, L1W, L1b, L2W, L2b):
    c = L2W.shape[1]
    return pl.pallas_call(
        _tc_c_body,
        out_shape=jax.ShapeDtypeStruct((_G, c), jnp.float32),
    )(part2, W2, b2row, batch2, L1W, L1b, L2W, L2b)


def kernel(x, edge_index, batch, W1, b1, W2, b2, L1W, L1b, L2W, L2b):
    e3 = edge_index.reshape(2, _ROWS, _CH)
    batch2 = batch.reshape(1, _N)
    b1row = b1.reshape(1, -1)

    hist, src_sc, dst_sc = _sc_deg(e3)
    xw = _tc_xw(x, W1)
    part1 = _sc_agg1(src_sc, dst_sc, hist, xw)
    part2 = _sc_agg2(src_sc, dst_sc, hist, xw, part1, b1row)
    return _tc_c(part2, W2, b2.reshape(1, -1), batch2,
                 L1W, L1b.reshape(1, -1), L2W, L2b.reshape(1, -1))


# speedup vs baseline: 1.1478x; 1.0099x over previous
import functools

import jax
import jax.numpy as jnp
from jax import lax
from jax.experimental import pallas as pl
from jax.experimental.pallas import tpu as pltpu
from jax.experimental.pallas import tpu_sc as plsc

_N = 10000
_E = 320000
_G = 8
_D = 16

_CH = 125
_ROWS = _E // _CH
_RPT = _ROWS // 32
_NSL = _N // 16

_mesh = plsc.VectorSubcoreMesh(core_axis_name="c", subcore_axis_name="s")
_sc_params = pltpu.CompilerParams(use_tc_tiling_on_sc=False,
                                  needs_layout_passes=False)


def _fill_rows(buf, val):
    @pl.loop(0, _CH, step=5)
    def _(r0_):
        for dd in range(5):
            buf[r0_ + dd, pl.ds(0, _D)] = jnp.full((_D,), val, jnp.float32)


def _zero_acc(zbuf, acc, s):
    @pl.loop(0, _NSL // _CH)
    def _(kk):
        pltpu.sync_copy(zbuf, acc.at[pl.ds(s * _NSL + kk * _CH, _CH)])


def _copy_out(acc, out_h, c, s):
    off = pl.multiple_of(jnp.minimum(s * 632, _N - 632), 8)
    pltpu.sync_copy(acc.at[pl.ds(off, 632)], out_h.at[c, pl.ds(off, 632)])


def _rsqrt16(d):
    i = plsc.bitcast(d, jnp.int32)
    i = jnp.int32(0x5F3759DF) - lax.shift_right_logical(i, 1)
    y = plsc.bitcast(i, jnp.float32)
    for _ in range(3):
        y = y * (1.5 - 0.5 * d * y * y)
    return y


def _agg_pipeline(table_sp, acc, sidx, didx, rows, gsem, ssem):
    def g_start(j, b):
        pltpu.async_copy(table_sp.at[sidx.at[j]], rows[b], gsem[b])

    def g_wait(j, b):
        pltpu.make_async_copy(table_sp.at[sidx.at[j]], rows[b],
                              gsem[b]).wait()

    def s_start(j, b):
        pltpu.async_copy(rows[b], acc.at[didx.at[j]], ssem[b], add=True)

    def s_wait(j, b):
        pltpu.make_async_copy(rows[b], acc.at[didx.at[j]], ssem[b]).wait()

    for b in range(4):
        g_start(b, b)

    @pl.loop(0, _RPT - 4, step=4)
    def _(j):
        for b in range(4):
            g_wait(j + b, b)
            s_start(j + b, b)
        for b in range(4):
            s_wait(j + b, b)
            g_start(j + 4 + b, b)

    for b in range(4):
        g_wait(_RPT - 4 + b, b)
        s_start(_RPT - 4 + b, b)
    for b in range(4):
        s_wait(_RPT - 4 + b, b)


def _sc_deg(e3):
    @functools.partial(
        pl.kernel,
        out_type=[
            jax.ShapeDtypeStruct((2, _N, _D), jnp.float32),
            jax.ShapeDtypeStruct((_ROWS, _CH), jnp.int32),
            jax.ShapeDtypeStruct((_ROWS, _CH), jnp.int32),
        ],
        mesh=_mesh,
        compiler_params=_sc_params,
        scratch_types=[
            pltpu.VMEM((_RPT, _CH), jnp.int32),
            pltpu.VMEM((_RPT, _CH), jnp.int32),
            pltpu.VMEM((_CH, _D), jnp.float32),
            pltpu.VMEM_SHARED((_N, _D), jnp.float32),
            pltpu.SemaphoreType.DMA,
        ],
    )
    def k(e_h, hist_h, srco_h, dsto_h, sidx, didx, ones, acc, sem):
        c = lax.axis_index("c")
        s = lax.axis_index("s")
        w = c * 16 + s
        _fill_rows(ones, 0.0)
        _zero_acc(ones, acc, s)
        _fill_rows(ones, 1.0)
        pltpu.sync_copy(e_h.at[0, pl.ds(w * _RPT, _RPT)], sidx)
        pltpu.sync_copy(e_h.at[1, pl.ds(w * _RPT, _RPT)], didx)
        pltpu.sync_copy(sidx, srco_h.at[pl.ds(w * _RPT, _RPT)])
        pltpu.sync_copy(didx, dsto_h.at[pl.ds(w * _RPT, _RPT)])
        plsc.subcore_barrier()

        @pl.loop(0, 8)
        def _(j):
            pltpu.async_copy(ones, acc.at[didx.at[j]], sem, add=True)

        @pl.loop(8, _RPT, step=8)
        def _(j):
            @pl.loop(0, 8)
            def _(b):
                pltpu.async_copy(ones, acc.at[didx.at[j + b]], sem, add=True)
            @pl.loop(0, 8)
            def _(b):
                pltpu.make_async_copy(ones, acc.at[didx.at[j - 8 + b]],
                                      sem).wait()

        @pl.loop(_RPT - 8, _RPT)
        def _(j):
            pltpu.make_async_copy(ones, acc.at[didx.at[j]], sem).wait()

        plsc.subcore_barrier()
        _copy_out(acc, hist_h, c, s)

    return k(e3)


def _agg_scratch(n_prologue_bufs):
    return (
        [
            pltpu.VMEM((_RPT, _CH), jnp.int32),
            pltpu.VMEM((_RPT, _CH), jnp.int32),
            pltpu.VMEM((_CH, _D), jnp.float32),
            pltpu.VMEM((_CH, _D), jnp.float32),
            pltpu.VMEM((_CH, _D), jnp.float32),
            pltpu.VMEM((_CH, _D), jnp.float32),
            pltpu.VMEM((_CH, _D), jnp.float32),
        ]
        + [pltpu.VMEM((_NSL, _D), jnp.float32)
           for _ in range(n_prologue_bufs)]
        + [
            pltpu.VMEM_SHARED((_N, _D), jnp.float32),
            pltpu.VMEM_SHARED((_N, _D), jnp.float32),
        ]
        + [pltpu.SemaphoreType.DMA for _ in range(8)]
    )


def _sc_agg1(src_sc, dst_sc, hist, xw):
    @functools.partial(
        pl.kernel,
        out_type=jax.ShapeDtypeStruct((2, _N, _D), jnp.float32),
        mesh=_mesh,
        compiler_params=_sc_params,
        scratch_types=_agg_scratch(3),
    )
    def k(src_h, dst_h, hist_h, xw_h, out_h, sidx, didx, zbuf,
          r0, r1, r2, r3, h0b, h1b, xwb, table, acc,
          g0, g1, g2, g3, s0, s1, s2, s3):
        c = lax.axis_index("c")
        s = lax.axis_index("s")
        w = c * 16 + s
        base = s * _NSL
        pltpu.sync_copy(hist_h.at[0, pl.ds(base, _NSL)], h0b)
        pltpu.sync_copy(hist_h.at[1, pl.ds(base, _NSL)], h1b)
        pltpu.sync_copy(xw_h.at[pl.ds(base, _NSL)], xwb)
        pltpu.sync_copy(src_h.at[pl.ds(w * _RPT, _RPT)], sidx)
        pltpu.sync_copy(dst_h.at[pl.ds(w * _RPT, _RPT)], didx)

        @pl.loop(0, _NSL, step=5)
        def _(r0_):
            for dd in range(5):
                r = r0_ + dd
                d16 = h0b[r, pl.ds(0, _D)] + h1b[r, pl.ds(0, _D)] + 1.0
                y = _rsqrt16(d16)
                xwb[r, pl.ds(0, _D)] = y * xwb[r, pl.ds(0, _D)]

        pltpu.sync_copy(xwb, table.at[pl.ds(base, _NSL)])
        _fill_rows(zbuf, 0.0)
        _zero_acc(zbuf, acc, s)
        plsc.subcore_barrier()
        _agg_pipeline(table, acc, sidx, didx, (r0, r1, r2, r3),
                      (g0, g1, g2, g3), (s0, s1, s2, s3))
        plsc.subcore_barrier()
        _copy_out(acc, out_h, c, s)

    return k(src_sc, dst_sc, hist, xw)


def _sc_agg2(src_sc, dst_sc, hist, xw, part1, b1row):
    @functools.partial(
        pl.kernel,
        out_type=jax.ShapeDtypeStruct((2, _N, _D), jnp.float32),
        mesh=_mesh,
        compiler_params=_sc_params,
        scratch_types=_agg_scratch(5) + [pltpu.VMEM((1, _D), jnp.float32)],
    )
    def k(src_h, dst_h, hist_h, xw_h, p1_h, b1_h, out_h, sidx, didx,
          zbuf, r0, r1, r2, r3, h0b, h1b, xwb, p0b, p1b, table, acc,
          g0, g1, g2, g3, s0, s1, s2, s3, b1b):
        c = lax.axis_index("c")
        s = lax.axis_index("s")
        w = c * 16 + s
        base = s * _NSL
        pltpu.sync_copy(hist_h.at[0, pl.ds(base, _NSL)], h0b)
        pltpu.sync_copy(hist_h.at[1, pl.ds(base, _NSL)], h1b)
        pltpu.sync_copy(xw_h.at[pl.ds(base, _NSL)], xwb)
        pltpu.sync_copy(p1_h.at[0, pl.ds(base, _NSL)], p0b)
        pltpu.sync_copy(p1_h.at[1, pl.ds(base, _NSL)], p1b)
        pltpu.sync_copy(b1_h, b1b)
        pltpu.sync_copy(src_h.at[pl.ds(w * _RPT, _RPT)], sidx)
        pltpu.sync_copy(dst_h.at[pl.ds(w * _RPT, _RPT)], didx)
        b1v = b1b[0, pl.ds(0, _D)]

        @pl.loop(0, _NSL, step=5)
        def _(r0_):
            for dd in range(5):
                r = r0_ + dd
                d16 = h0b[r, pl.ds(0, _D)] + h1b[r, pl.ds(0, _D)] + 1.0
                y = _rsqrt16(d16)
                u1 = y * xwb[r, pl.ds(0, _D)]
                h1 = (y * (p0b[r, pl.ds(0, _D)] + p1b[r, pl.ds(0, _D)] + u1)
                      + b1v)
                v = y * jnp.maximum(h1, 0.0)
                xwb[r, pl.ds(0, _D)] = v
                p1b[r, pl.ds(0, _D)] = 0.5 * v

        pltpu.sync_copy(xwb, table.at[pl.ds(base, _NSL)])
        pltpu.sync_copy(p1b, acc.at[pl.ds(base, _NSL)])
        plsc.subcore_barrier()
        _agg_pipeline(table, acc, sidx, didx, (r0, r1, r2, r3),
                      (g0, g1, g2, g3), (s0, s1, s2, s3))
        plsc.subcore_barrier()
        pltpu.sync_copy(acc.at[pl.ds(base, _NSL)], p0b)

        @pl.loop(0, _NSL, step=5)
        def _(r0_):
            for dd in range(5):
                r = r0_ + dd
                d16 = h0b[r, pl.ds(0, _D)] + h1b[r, pl.ds(0, _D)] + 1.0
                y = _rsqrt16(d16)
                p0b[r, pl.ds(0, _D)] = y * p0b[r, pl.ds(0, _D)]

        pltpu.sync_copy(p0b, acc.at[pl.ds(base, _NSL)])
        plsc.subcore_barrier()
        _copy_out(acc, out_h, c, s)

    return k(src_sc, dst_sc, hist, xw, part1, b1row)


def _tc_xw_body(x_ref, w_ref, o_ref):
    o_ref[...] = jnp.dot(x_ref[...], w_ref[...],
                         preferred_element_type=jnp.float32)


def _tc_xw(x, W1):
    f_in, h1 = W1.shape
    return pl.pallas_call(
        _tc_xw_body,
        out_shape=jax.ShapeDtypeStruct((_N, h1), jnp.float32),
    )(x, W1)


def _tc_c_body(p2_ref, w2_ref, b2_ref,
               bat_ref, l1w_ref, l1b_ref, l2w_ref, l2b_ref, out_ref):
    t = p2_ref[0] + p2_ref[1]
    h = jnp.dot(t, w2_ref[...],
                preferred_element_type=jnp.float32) + b2_ref[...]
    h = jnp.maximum(h, 0.0)
    hc = jnp.concatenate([h, jnp.ones((_N, 1), jnp.float32)], axis=1)
    onehot = (bat_ref[...] == lax.broadcasted_iota(jnp.int32, (_G, 1), 0))
    m = onehot.astype(jnp.float32)
    sums = lax.dot_general(m, hc, (((1,), (0,)), ((), ())),
                           preferred_element_type=jnp.float32)
    h2 = sums.shape[1] - 1
    cnt = sums[:, h2:h2 + 1]
    pooled = sums[:, 0:h2] / jnp.maximum(cnt, 1.0)
    z = jnp.dot(pooled, l1w_ref[...],
                preferred_element_type=jnp.float32) + l1b_ref[...]
    z = jnp.maximum(z, 0.0)
    z = jnp.dot(z, l2w_ref[...],
                preferred_element_type=jnp.float32) + l2b_ref[...]
    mx = jnp.max(z, axis=1, keepdims=True)
    lse = mx + jnp.log(jnp.sum(jnp.exp(z - mx), axis=1, keepdims=True))
    out_ref[...] = z - lse


def _tc_c(part2, W2, b2row, batch2, L1W, L1b, L2W, L2b):
    c = L2W.shape[1]
    return pl.pallas_call(
        _tc_c_body,
        out_shape=jax.ShapeDtypeStruct((_G, c), jnp.float32),
    )(part2, W2, b2row, batch2, L1W, L1b, L2W, L2b)


def kernel(x, edge_index, batch, W1, b1, W2, b2, L1W, L1b, L2W, L2b):
    e3 = edge_index.reshape(2, _ROWS, _CH)
    batch2 = batch.reshape(1, _N)
    b1row = b1.reshape(1, -1)

    hist, src_sc, dst_sc = _sc_deg(e3)
    xw = _tc_xw(x, W1)
    part1 = _sc_agg1(src_sc, dst_sc, hist, xw)
    part2 = _sc_agg2(src_sc, dst_sc, hist, xw, part1, b1row)
    return _tc_c(part2, W2, b2.reshape(1, -1), batch2,
                 L1W, L1b.reshape(1, -1), L2W, L2b.reshape(1, -1))
